# Initial kernel scaffold; baseline (speedup 1.0000x reference)
import functools

import jax
import jax.numpy as jnp
from jax import lax
from jax.experimental import pallas as pl
from jax.experimental.pallas import tpu as pltpu
from jax.experimental.pallas import tpu_sc as plsc

BATCH = 16384
HIST = 200
EMBED_DIM = 32
TOTAL = BATCH * HIST

NUM_CORES = 2
NUM_SUBCORES = 16
NUM_WORKERS = NUM_CORES * NUM_SUBCORES
PER_WORKER = TOTAL // NUM_WORKERS
CHUNK = 2048
NUM_CHUNKS = PER_WORKER // CHUNK

_MESH = plsc.VectorSubcoreMesh(core_axis_name="c", subcore_axis_name="s")


@functools.partial(
    pl.kernel,
    mesh=_MESH,
    out_type=jax.ShapeDtypeStruct((TOTAL, EMBED_DIM), jnp.float32),
    scratch_types=[
        pltpu.VMEM((CHUNK,), jnp.int32),
        pltpu.VMEM((CHUNK, EMBED_DIM), jnp.float32),
        pltpu.SemaphoreType.DMA,
    ],
)
def _sc_embed(table_hbm, idx_hbm, out_hbm, idx_v, rows_v, sem):
    wid = lax.axis_index("s") * NUM_CORES + lax.axis_index("c")
    base = wid * PER_WORKER

    def step(g, _):
        off = base + g * CHUNK
        pltpu.sync_copy(idx_hbm.at[pl.ds(off, CHUNK)], idx_v)
        pltpu.async_copy(table_hbm.at[idx_v], rows_v, sem).wait()
        pltpu.sync_copy(rows_v, out_hbm.at[pl.ds(off, CHUNK)])
        return 0

    lax.fori_loop(0, NUM_CHUNKS, step, 0)


def kernel(info, table):
    idx = info.reshape(TOTAL).astype(jnp.int32)
    out = _sc_embed(table, idx)
    return out.reshape(BATCH, HIST, EMBED_DIM)


# speedup vs baseline: 4.9399x; 4.9399x over previous
import functools

import jax
import jax.numpy as jnp
from jax import lax
from jax.experimental import pallas as pl
from jax.experimental.pallas import tpu as pltpu
from jax.experimental.pallas import tpu_sc as plsc

BATCH = 16384
HIST = 200
EMBED_DIM = 32
TOTAL = BATCH * HIST

NUM_CORES = 2
NUM_SUBCORES = 16
NUM_WORKERS = NUM_CORES * NUM_SUBCORES
PER_WORKER = TOTAL // NUM_WORKERS
CHUNK = 2048
NUM_CHUNKS = PER_WORKER // CHUNK

_MESH = plsc.VectorSubcoreMesh(core_axis_name="c", subcore_axis_name="s")


@functools.partial(
    pl.kernel,
    mesh=_MESH,
    out_type=jax.ShapeDtypeStruct((TOTAL, EMBED_DIM), jnp.float32),
    scratch_types=[
        pltpu.VMEM((CHUNK,), jnp.int32),
        pltpu.VMEM((CHUNK, EMBED_DIM), jnp.float32),
        pltpu.SemaphoreType.DMA,
    ],
    compiler_params=pltpu.CompilerParams(use_tc_tiling_on_sc=False),
)
def _sc_embed(table_hbm, idx_hbm, out_hbm, idx_v, rows_v, sem):
    wid = lax.axis_index("s") * NUM_CORES + lax.axis_index("c")
    base = wid * PER_WORKER

    def step(g, _):
        off = base + g * CHUNK
        pltpu.sync_copy(idx_hbm.at[pl.ds(off, CHUNK)], idx_v)
        pltpu.async_copy(table_hbm.at[idx_v], rows_v, sem).wait()
        pltpu.sync_copy(rows_v, out_hbm.at[pl.ds(off, CHUNK)])
        return 0

    lax.fori_loop(0, NUM_CHUNKS, step, 0)


def kernel(info, table):
    idx = info.reshape(TOTAL).astype(jnp.int32)
    out = _sc_embed(table, idx)
    return out.reshape(BATCH, HIST, EMBED_DIM)


# speedup vs baseline: 5.0387x; 1.0200x over previous
import functools

import jax
import jax.numpy as jnp
from jax import lax
from jax.experimental import pallas as pl
from jax.experimental.pallas import tpu as pltpu
from jax.experimental.pallas import tpu_sc as plsc

BATCH = 16384
HIST = 200
EMBED_DIM = 32
TOTAL = BATCH * HIST

NUM_CORES = 2
NUM_SUBCORES = 16
NUM_WORKERS = NUM_CORES * NUM_SUBCORES
PER_WORKER = TOTAL // NUM_WORKERS
NBUF = 4
CHUNK = 512
NUM_CHUNKS = PER_WORKER // CHUNK
NROUNDS = NUM_CHUNKS // NBUF

_MESH = plsc.VectorSubcoreMesh(core_axis_name="c", subcore_axis_name="s")


@functools.partial(
    pl.kernel,
    mesh=_MESH,
    out_type=jax.ShapeDtypeStruct((TOTAL, EMBED_DIM), jnp.float32),
    scratch_types=[
        pltpu.VMEM((NBUF, CHUNK), jnp.int32),
        pltpu.VMEM((NBUF, CHUNK, EMBED_DIM), jnp.float32),
        pltpu.SemaphoreType.DMA((NBUF,)),
        pltpu.SemaphoreType.DMA((NBUF,)),
        pltpu.SemaphoreType.DMA((NBUF,)),
    ],
    compiler_params=pltpu.CompilerParams(use_tc_tiling_on_sc=False),
)
def _sc_embed(table_hbm, idx_hbm, out_hbm, idx_v, rows_v, idx_sem, gat_sem,
              st_sem):
    wid = lax.axis_index("s") * NUM_CORES + lax.axis_index("c")
    base = wid * PER_WORKER

    def idx_load(b, off):
        pltpu.async_copy(idx_hbm.at[pl.ds(off, CHUNK)], idx_v.at[b],
                         idx_sem.at[b])

    def idx_wait(b, off):
        pltpu.make_async_copy(idx_hbm.at[pl.ds(off, CHUNK)], idx_v.at[b],
                              idx_sem.at[b]).wait()

    def gather(b):
        pltpu.async_copy(table_hbm.at[idx_v.at[b]], rows_v.at[b],
                         gat_sem.at[b])

    def gather_wait(b):
        pltpu.make_async_copy(table_hbm.at[idx_v.at[b]], rows_v.at[b],
                              gat_sem.at[b]).wait()

    def store(b, off):
        pltpu.async_copy(rows_v.at[b], out_hbm.at[pl.ds(off, CHUNK)],
                         st_sem.at[b])

    def store_wait(b, off):
        pltpu.make_async_copy(rows_v.at[b], out_hbm.at[pl.ds(off, CHUNK)],
                              st_sem.at[b]).wait()

    for b in range(NBUF):
        idx_load(b, base + b * CHUNK)
    for b in range(NBUF):
        idx_wait(b, base + b * CHUNK)
        gather(b)
    for b in range(NBUF):
        off = base + b * CHUNK
        gather_wait(b)
        store(b, off)
        idx_load(b, off + NBUF * CHUNK)

    def round_body(t, issue_next):
        off0 = base + t * (NBUF * CHUNK)
        for b in range(NBUF):
            off = off0 + b * CHUNK
            store_wait(b, off)
            idx_wait(b, off)
            gather(b)
        for b in range(NBUF):
            off = off0 + b * CHUNK
            gather_wait(b)
            store(b, off)
            if issue_next:
                idx_load(b, off + NBUF * CHUNK)

    lax.fori_loop(1, NROUNDS - 1,
                  lambda t, c: (round_body(t, True), c)[1], 0)
    round_body(NROUNDS - 1, False)

    for b in range(NBUF):
        store_wait(b, base + (NROUNDS - 1) * (NBUF * CHUNK) + b * CHUNK)


def kernel(info, table):
    idx = info.reshape(TOTAL).astype(jnp.int32)
    out = _sc_embed(table, idx)
    return out.reshape(BATCH, HIST, EMBED_DIM)
